# Initial kernel scaffold; baseline (speedup 1.0000x reference)
import jax
import jax.numpy as jnp
from jax.experimental import pallas as pl


def kernel(x, Win, b1, V, b_rec, Wlsm):
    raise NotImplementedError("write your pallas kernel here")


# speedup vs baseline: 6.6015x; 6.6015x over previous
import jax
import jax.numpy as jnp
import numpy as np
from jax.experimental import pallas as pl
from jax.experimental.pallas import tpu as pltpu

_N = 2048
_IN = 512
_T = 32
_ALPHA = 0.9
_BETA = 0.9
_TH = 20.0
_DELTA = float((np.exp(-1.0 / 20.0) - np.exp(1.0 / 20.0)) * 0.001)

_WBLK = 256


def _scan_kernel(x_ref, winT_ref, b1_ref, vT_ref, brec_ref, spk_ref, curr_ref):
    curr_ref[:] = (
        jax.lax.dot_general(
            x_ref[:], winT_ref[:], (((1,), (0,)), ((), ())),
            preferred_element_type=jnp.float32,
        )
        + b1_ref[:]
    )

    z = jnp.zeros((1, _N), jnp.float32)

    def body(t, carry):
        spk, syn, mem = carry
        rec = (
            jax.lax.dot_general(
                spk, vT_ref[:], (((1,), (0,)), ((), ())),
                preferred_element_type=jnp.float32,
            )
            + brec_ref[:]
        )
        syn = _ALPHA * syn + curr_ref[pl.ds(t, 1), :] + rec
        mem = _BETA * mem + syn - spk * _TH
        spk = ((mem - _TH) > 0.0).astype(jnp.float32)
        spk_ref[pl.ds(t, 1), :] = spk
        return (spk, syn, mem)

    jax.lax.fori_loop(0, _T, body, (z, z, z), unroll=False)


def _stdp_kernel(sT_ref, s_ref, w_ref, out_ref):
    st1 = sT_ref[:, 1:_T]
    s0 = s_ref[0 : _T - 1, :]
    c = jax.lax.dot_general(
        st1, s0, (((1,), (0,)), ((), ())), preferred_element_type=jnp.float32
    )
    out_ref[:] = jnp.clip(w_ref[:] + _DELTA * c, -1.0, 1.0)


def kernel(x, Win, b1, V, b_rec, Wlsm):
    x2 = x.reshape(_T, _IN)
    winT = Win.T
    vT = V.T
    b1r = b1.reshape(1, _N)
    brr = b_rec.reshape(1, _N)

    spk_rec = pl.pallas_call(
        _scan_kernel,
        out_shape=jax.ShapeDtypeStruct((_T, _N), jnp.float32),
        scratch_shapes=[pltpu.VMEM((_T, _N), jnp.float32)],
    )(x2, winT, b1r, vT, brr)

    spk_recT = spk_rec.T

    nblk = _N // _WBLK
    w_final = pl.pallas_call(
        _stdp_kernel,
        grid=(nblk,),
        in_specs=[
            pl.BlockSpec((_WBLK, _T), lambda i: (i, 0)),
            pl.BlockSpec((_T, _N), lambda i: (0, 0)),
            pl.BlockSpec((_WBLK, _N), lambda i: (i, 0)),
        ],
        out_specs=pl.BlockSpec((_WBLK, _N), lambda i: (i, 0)),
        out_shape=jax.ShapeDtypeStruct((_N, _N), jnp.float32),
    )(spk_recT, spk_rec, Wlsm)

    return spk_rec.reshape(_T, 1, _N), w_final
